# Initial kernel scaffold; baseline (speedup 1.0000x reference)
import jax
import jax.numpy as jnp
from jax.experimental import pallas as pl


def kernel(visual_embed, textual_embed, part_embed, attribute_embed, seg_feat, W, labels, masks, vmask, tmask):
    raise NotImplementedError("write your pallas kernel here")


# speedup vs baseline: 5.4407x; 5.4407x over previous
import functools

import jax
import jax.numpy as jnp
from jax.experimental import pallas as pl
from jax.experimental.pallas import tpu as pltpu

B = 1024
P = 5
D = 128
NC = 11003
SCC = 6
H = 48
WD = 16
SCALE = 28.0
TOPK = 8
NEG = -3.0e38

NCP = 11264
BK = 1408


def _softplus_pt(s):
    return jnp.log1p(jnp.exp(-10.0 * (s - 0.6)))


def _softplus_nt(s):
    return jnp.log1p(jnp.exp(40.0 * (s - 0.4)))


def _l2n(x):
    n = jnp.sqrt(jnp.sum(x * x, axis=1, keepdims=True))
    return x / jnp.maximum(n, 1e-12)


def _inst_kernel(w_ref, v_ref, t_ref, labc_ref, out_ref, accv, acct, pkv, pkt):
    j = pl.program_id(0)
    nb = pl.num_programs(0)

    @pl.when(j == 0)
    def _():
        accv[...] = jnp.zeros_like(accv)
        acct[...] = jnp.zeros_like(acct)
        pkv[...] = jnp.zeros_like(pkv)
        pkt[...] = jnp.zeros_like(pkt)

    wb = w_ref[...]
    n2 = jnp.sum(wb * wb, axis=0, keepdims=True)
    inv = 1.0 / jnp.maximum(jnp.sqrt(n2), 1e-12)
    col = jax.lax.broadcasted_iota(jnp.int32, (1, BK), 1) + j * BK
    valid = col < NC
    labc = labc_ref[...]
    pickm = labc == col

    vn = _l2n(v_ref[...])
    tn = _l2n(t_ref[...])
    lv = SCALE * jax.lax.dot_general(
        vn, wb, (((1,), (0,)), ((), ())),
        preferred_element_type=jnp.float32,
        precision=jax.lax.Precision.HIGHEST) * inv
    lt = SCALE * jax.lax.dot_general(
        tn, wb, (((1,), (0,)), ((), ())),
        preferred_element_type=jnp.float32,
        precision=jax.lax.Precision.HIGHEST) * inv
    accv[...] += jnp.sum(jnp.where(valid, jnp.exp(lv), 0.0), axis=1,
                         keepdims=True)
    acct[...] += jnp.sum(jnp.where(valid, jnp.exp(lt), 0.0), axis=1,
                         keepdims=True)
    pkv[...] += jnp.sum(jnp.where(pickm, lv, 0.0), axis=1, keepdims=True)
    pkt[...] += jnp.sum(jnp.where(pickm, lt, 0.0), axis=1, keepdims=True)

    @pl.when(j == nb - 1)
    def _():
        lossv = jnp.sum(jnp.log(accv[...]) - pkv[...])
        losst = jnp.sum(jnp.log(acct[...]) - pkt[...])
        out_ref[...] = ((lossv + losst) / B).reshape(1, 1)


def _instance_loss(visual, textual, w, labels):
    wp = jnp.pad(w, ((0, 0), (0, NCP - NC)))
    labc = labels.astype(jnp.int32).reshape(B, 1)
    grid = NCP // BK
    return pl.pallas_call(
        _inst_kernel,
        grid=(grid,),
        in_specs=[
            pl.BlockSpec((D, BK), lambda j: (0, j)),
            pl.BlockSpec((B, D), lambda j: (0, 0)),
            pl.BlockSpec((B, D), lambda j: (0, 0)),
            pl.BlockSpec((B, 1), lambda j: (0, 0)),
        ],
        out_specs=pl.BlockSpec((1, 1), lambda j: (0, 0)),
        out_shape=jax.ShapeDtypeStruct((1, 1), jnp.float32),
        scratch_shapes=[pltpu.VMEM((B, 1), jnp.float32)] * 4,
    )(wp, visual, textual, labc)[0, 0]


_RB = 256


def _mask_kernel(x_ref, m_ref, out_ref, acc):
    j = pl.program_id(0)
    nb = pl.num_programs(0)

    @pl.when(j == 0)
    def _():
        acc[...] = jnp.zeros_like(acc)

    x = x_ref[...]
    m = m_ref[...]
    se = jnp.sum(jnp.exp(x), axis=1)
    lse = jnp.log(se)
    picked = jnp.zeros_like(lse)
    for c in range(SCC):
        picked += jnp.where(m == c, x[:, c, :], 0.0)
    acc[...] += jnp.sum(lse - picked).reshape(1, 1)

    @pl.when(j == nb - 1)
    def _():
        out_ref[...] = P * acc[...] / (B * P * H * WD)


def _mask_loss(seg_feat, masks):
    seg = seg_feat.reshape(B * P, SCC, H * WD)
    msk = masks.astype(jnp.int32).reshape(B * P, H * WD)
    grid = (B * P) // _RB
    return pl.pallas_call(
        _mask_kernel,
        grid=(grid,),
        in_specs=[
            pl.BlockSpec((_RB, SCC, H * WD), lambda j: (j, 0, 0)),
            pl.BlockSpec((_RB, H * WD), lambda j: (j, 0)),
        ],
        out_specs=pl.BlockSpec((1, 1), lambda j: (0, 0)),
        out_shape=jax.ShapeDtypeStruct((1, 1), jnp.float32),
        scratch_shapes=[pltpu.VMEM((1, 1), jnp.float32)],
    )(seg, msk)[0, 0]


_CH = 128


def _pick_f32(vec, idx, lane):
    return jnp.sum(jnp.where(lane == idx, vec, 0.0))


def _pick_i32(vec, idx, lane):
    return jnp.sum(jnp.where(lane == idx, vec, 0))


def _top8(vec, lane):
    idxs = []
    cur = vec
    for _ in range(TOPK):
        mx = jnp.max(cur)
        idx = jnp.min(jnp.where(cur == mx, lane, B))
        idxs.append(idx)
        cur = jnp.where(lane == idx, NEG, cur)
    return idxs


def _rank_lt8(vec, me, lane):
    x = jnp.sum(jnp.where(lane == me, vec, 0.0))
    gt = jnp.sum((vec > x).astype(jnp.float32))
    eq = jnp.sum(jnp.where((lane < me) & (vec == x), 1.0, 0.0))
    return ((gt + eq) < TOPK).astype(jnp.float32)


def _align_kernel(x_ref, y_ref, labc_ref, labr_ref, pmf_ref, amf_ref,
                  gout_ref, lout_ref, s_sc, st_sc, g_acc, l_acc):
    m = pl.program_id(0)

    @pl.when(m == 0)
    def _():
        g_acc[...] = jnp.zeros_like(g_acc)
        l_acc[...] = jnp.zeros_like(l_acc)

    xn = _l2n(x_ref[0])
    yn = _l2n(y_ref[0])
    s_sc[...] = jax.lax.dot_general(
        xn, yn, (((1,), (1,)), ((), ())),
        preferred_element_type=jnp.float32,
        precision=jax.lax.Precision.HIGHEST)
    st_sc[...] = jax.lax.dot_general(
        yn, xn, (((1,), (1,)), ((), ())),
        preferred_element_type=jnp.float32,
        precision=jax.lax.Precision.HIGHEST)

    labr = labr_ref[...]
    lane = jax.lax.broadcasted_iota(jnp.int32, (1, B), 1)
    part = jnp.maximum(m - 1, 0)
    is_g = (m == 0)

    p_row = pmf_ref[pl.ds(part, 1), :]
    a_row = amf_ref[pl.ds(part, 1), :]
    one = jnp.ones((1, B), jnp.float32)
    colw = jnp.where(is_g, one, a_row * (1.0 + p_row))
    roww = jnp.where(is_g, one, p_row)

    base = jnp.zeros((1, 1), jnp.float32)
    for r in range(B // _CH):
        sb = s_sc[r * _CH:(r + 1) * _CH, :]
        labc = labc_ref[r * _CH:(r + 1) * _CH, :]
        pos = (labc == labr).astype(jnp.float32)
        pt = _softplus_pt(sb)
        nt = _softplus_nt(sb)
        inner = (nt + pos * (pt - nt)) * colw
        q = jnp.sum(inner, axis=1, keepdims=True)
        base += jax.lax.dot_general(
            roww[:, r * _CH:(r + 1) * _CH], q,
            (((1,), (0,)), ((), ())),
            preferred_element_type=jnp.float32,
            precision=jax.lax.Precision.HIGHEST)

    @pl.when(is_g)
    def _():
        g_acc[...] += 2.0 * base / B

    @pl.when(m > 0)
    def _():
        r_i = s_sc[pl.ds(part, 1), :]
        c_i = st_sc[pl.ds(part, 1), :]
        f1 = _top8(r_i, lane)
        f2 = _top8(c_i, lane)
        corr = jnp.zeros((), jnp.float32)
        for k in range(TOPK):
            colv = st_sc[pl.ds(f1[k], 1), :]
            hit = _rank_lt8(colv, part, lane)
            lab_f = _pick_i32(labr, f1[k], lane)
            a_f = _pick_f32(a_row, f1[k], lane)
            poseq = (labr == lab_f).astype(jnp.float32)
            d = _softplus_pt(colv) - _softplus_nt(colv)
            corr += hit * a_f * jnp.sum(p_row * (1.0 - poseq) * d)
            rowv = s_sc[pl.ds(f2[k], 1), :]
            hit2 = _rank_lt8(rowv, part, lane)
            lab_f2 = _pick_i32(labr, f2[k], lane)
            p_f2 = _pick_f32(p_row, f2[k], lane)
            poseq2 = (labr == lab_f2).astype(jnp.float32)
            d2 = _softplus_pt(rowv) - _softplus_nt(rowv)
            corr += hit2 * p_f2 * jnp.sum(p_row * a_row * (1.0 - poseq2) * d2)
        l_acc[...] += (base + corr) / B

    @pl.when(m == P)
    def _():
        gout_ref[...] = g_acc[...]
        lout_ref[...] = l_acc[...] / P


def _align_losses(visual, textual, part_embed, attribute_embed,
                  labels, vmask, tmask):
    xs = jnp.concatenate([visual[None], part_embed], axis=0)
    ys = jnp.concatenate([textual[None], attribute_embed], axis=0)
    labc = labels.astype(jnp.int32).reshape(B, 1)
    labr = labels.astype(jnp.int32).reshape(1, B)
    pmfT = vmask.astype(jnp.float32).T
    amfT = tmask.astype(jnp.float32).T
    gout, lout = pl.pallas_call(
        _align_kernel,
        grid=(P + 1,),
        in_specs=[
            pl.BlockSpec((1, B, D), lambda m: (m, 0, 0)),
            pl.BlockSpec((1, B, D), lambda m: (m, 0, 0)),
            pl.BlockSpec((B, 1), lambda m: (0, 0)),
            pl.BlockSpec((1, B), lambda m: (0, 0)),
            pl.BlockSpec((P, B), lambda m: (0, 0)),
            pl.BlockSpec((P, B), lambda m: (0, 0)),
        ],
        out_specs=[
            pl.BlockSpec((1, 1), lambda m: (0, 0)),
            pl.BlockSpec((1, 1), lambda m: (0, 0)),
        ],
        out_shape=[
            jax.ShapeDtypeStruct((1, 1), jnp.float32),
            jax.ShapeDtypeStruct((1, 1), jnp.float32),
        ],
        scratch_shapes=[
            pltpu.VMEM((B, B), jnp.float32),
            pltpu.VMEM((B, B), jnp.float32),
            pltpu.VMEM((1, 1), jnp.float32),
            pltpu.VMEM((1, 1), jnp.float32),
        ],
    )(xs, ys, labc, labr, pmfT, amfT)
    return gout[0, 0], lout[0, 0]


def kernel(visual_embed, textual_embed, part_embed, attribute_embed,
           seg_feat, W, labels, masks, vmask, tmask):
    inst = _instance_loss(visual_embed, textual_embed, W, labels)
    mask = _mask_loss(seg_feat, masks)
    glob, loc = _align_losses(visual_embed, textual_embed, part_embed,
                              attribute_embed, labels, vmask, tmask)
    return jnp.stack([inst, mask, glob, loc])


# speedup vs baseline: 8.5456x; 1.5707x over previous
import functools

import jax
import jax.numpy as jnp
from jax.experimental import pallas as pl
from jax.experimental.pallas import tpu as pltpu

B = 1024
P = 5
D = 128
NC = 11003
SCC = 6
H = 48
WD = 16
SCALE = 28.0
TOPK = 8
NEG = -3.0e38

NCP = 11264
BK = 1408


def _softplus_pt(s):
    return jnp.log1p(jnp.exp(-10.0 * (s - 0.6)))


def _softplus_nt(s):
    return jnp.log1p(jnp.exp(40.0 * (s - 0.4)))


def _l2n(x):
    n = jnp.sqrt(jnp.sum(x * x, axis=1, keepdims=True))
    return x / jnp.maximum(n, 1e-12)


def _inst_kernel(w_ref, v_ref, t_ref, labc_ref, out_ref, accv, acct, pkv, pkt):
    j = pl.program_id(0)
    nb = pl.num_programs(0)

    @pl.when(j == 0)
    def _():
        accv[...] = jnp.zeros_like(accv)
        acct[...] = jnp.zeros_like(acct)
        pkv[...] = jnp.zeros_like(pkv)
        pkt[...] = jnp.zeros_like(pkt)

    wb = w_ref[...]
    n2 = jnp.sum(wb * wb, axis=0, keepdims=True)
    inv = 1.0 / jnp.maximum(jnp.sqrt(n2), 1e-12)
    col = jax.lax.broadcasted_iota(jnp.int32, (1, BK), 1) + j * BK
    valid = col < NC
    labc = labc_ref[...]
    pickm = labc == col

    vn = _l2n(v_ref[...])
    tn = _l2n(t_ref[...])
    lv = SCALE * jax.lax.dot_general(
        vn, wb, (((1,), (0,)), ((), ())),
        preferred_element_type=jnp.float32) * inv
    lt = SCALE * jax.lax.dot_general(
        tn, wb, (((1,), (0,)), ((), ())),
        preferred_element_type=jnp.float32) * inv
    accv[...] += jnp.sum(jnp.where(valid, jnp.exp(lv), 0.0), axis=1,
                         keepdims=True)
    acct[...] += jnp.sum(jnp.where(valid, jnp.exp(lt), 0.0), axis=1,
                         keepdims=True)
    pkv[...] += jnp.sum(jnp.where(pickm, lv, 0.0), axis=1, keepdims=True)
    pkt[...] += jnp.sum(jnp.where(pickm, lt, 0.0), axis=1, keepdims=True)

    @pl.when(j == nb - 1)
    def _():
        lossv = jnp.sum(jnp.log(accv[...]) - pkv[...])
        losst = jnp.sum(jnp.log(acct[...]) - pkt[...])
        out_ref[...] = ((lossv + losst) / B).reshape(1, 1)


def _instance_loss(visual, textual, w, labels):
    labc = labels.astype(jnp.int32).reshape(B, 1)
    grid = NCP // BK
    return pl.pallas_call(
        _inst_kernel,
        grid=(grid,),
        in_specs=[
            pl.BlockSpec((D, BK), lambda j: (0, j)),
            pl.BlockSpec((B, D), lambda j: (0, 0)),
            pl.BlockSpec((B, D), lambda j: (0, 0)),
            pl.BlockSpec((B, 1), lambda j: (0, 0)),
        ],
        out_specs=pl.BlockSpec((1, 1), lambda j: (0, 0)),
        out_shape=jax.ShapeDtypeStruct((1, 1), jnp.float32),
        scratch_shapes=[pltpu.VMEM((B, 1), jnp.float32)] * 4,
    )(w, visual, textual, labc)[0, 0]


_RB = 256
_HW = H * WD


def _mask_kernel(x_ref, m_ref, out_ref, acc):
    j = pl.program_id(0)
    nb = pl.num_programs(0)

    @pl.when(j == 0)
    def _():
        acc[...] = jnp.zeros_like(acc)

    m = m_ref[...]
    x0 = x_ref[:, 0:_HW]
    se = jnp.exp(x0)
    picked = jnp.where(m == 0, x0, 0.0)
    for c in range(1, SCC):
        xc = x_ref[:, c * _HW:(c + 1) * _HW]
        se += jnp.exp(xc)
        picked += jnp.where(m == c, xc, 0.0)
    acc[...] += jnp.sum(jnp.log(se) - picked).reshape(1, 1)

    @pl.when(j == nb - 1)
    def _():
        out_ref[...] = P * acc[...] / (B * P * H * WD)


def _mask_loss(seg_feat, masks):
    seg = seg_feat.reshape(B * P, SCC * _HW)
    msk = masks.reshape(B * P, _HW)
    grid = (B * P) // _RB
    return pl.pallas_call(
        _mask_kernel,
        grid=(grid,),
        in_specs=[
            pl.BlockSpec((_RB, SCC * _HW), lambda j: (j, 0)),
            pl.BlockSpec((_RB, _HW), lambda j: (j, 0)),
        ],
        out_specs=pl.BlockSpec((1, 1), lambda j: (0, 0)),
        out_shape=jax.ShapeDtypeStruct((1, 1), jnp.float32),
        scratch_shapes=[pltpu.VMEM((1, 1), jnp.float32)],
    )(seg, msk)[0, 0]


_CH = 128


def _pick_f32(vec, idx, lane):
    return jnp.sum(jnp.where(lane == idx, vec, 0.0))


def _pick_i32(vec, idx, lane):
    return jnp.sum(jnp.where(lane == idx, vec, 0))


def _top8(vec, lane):
    idxs = []
    cur = vec
    for _ in range(TOPK):
        mx = jnp.max(cur)
        idx = jnp.min(jnp.where(cur == mx, lane, B))
        idxs.append(idx)
        cur = jnp.where(lane == idx, NEG, cur)
    return idxs


def _align_kernel(vis_ref, txt_ref, pe_ref, ae_ref, labc_ref, labr_ref,
                  pmf_ref, amf_ref, gout_ref, lout_ref,
                  s_sc, st_sc, g_sc, g_acc, l_acc):
    m = pl.program_id(0)

    @pl.when(m == 0)
    def _():
        g_acc[...] = jnp.zeros_like(g_acc)
        l_acc[...] = jnp.zeros_like(l_acc)

    is_g0 = (m == 0)
    xn = _l2n(jnp.where(is_g0, vis_ref[...], pe_ref[0]))
    yn = _l2n(jnp.where(is_g0, txt_ref[...], ae_ref[0]))
    s_sc[...] = jax.lax.dot_general(
        xn, yn, (((1,), (1,)), ((), ())),
        preferred_element_type=jnp.float32,
        precision=jax.lax.Precision.HIGHEST)
    st_sc[...] = jax.lax.dot_general(
        yn, xn, (((1,), (1,)), ((), ())),
        preferred_element_type=jnp.float32,
        precision=jax.lax.Precision.HIGHEST)

    labr = labr_ref[...]
    lane = jax.lax.broadcasted_iota(jnp.int32, (1, B), 1)
    part = jnp.maximum(m - 1, 0)
    is_g = (m == 0)

    p_row = pmf_ref[pl.ds(part, 1), :]
    a_row = amf_ref[pl.ds(part, 1), :]
    one = jnp.ones((1, B), jnp.float32)
    colw = jnp.where(is_g, one, a_row * (1.0 + p_row))
    roww = jnp.where(is_g, one, p_row)

    base = jnp.zeros((1, 1), jnp.float32)
    for r in range(B // _CH):
        sb = s_sc[r * _CH:(r + 1) * _CH, :]
        labc = labc_ref[r * _CH:(r + 1) * _CH, :]
        pos = (labc == labr).astype(jnp.float32)
        pt = _softplus_pt(sb)
        nt = _softplus_nt(sb)
        inner = (nt + pos * (pt - nt)) * colw
        q = jnp.sum(inner, axis=1, keepdims=True)
        base += jax.lax.dot_general(
            roww[:, r * _CH:(r + 1) * _CH], q,
            (((1,), (0,)), ((), ())),
            preferred_element_type=jnp.float32,
            precision=jax.lax.Precision.HIGHEST)

    @pl.when(is_g)
    def _():
        g_acc[...] += 2.0 * base / B

    @pl.when(m > 0)
    def _():
        r_i = s_sc[pl.ds(part, 1), :]
        c_i = st_sc[pl.ds(part, 1), :]
        f1 = _top8(r_i, lane)
        f2 = _top8(c_i, lane)
        for k in range(TOPK):
            g_sc[k:k + 1, :] = st_sc[pl.ds(f1[k], 1), :]
            g_sc[TOPK + k:TOPK + k + 1, :] = s_sc[pl.ds(f2[k], 1), :]
        g = g_sc[...]
        d = _softplus_pt(g) - _softplus_nt(g)
        ridx = jax.lax.broadcasted_iota(jnp.int32, (2 * TOPK, 1), 0)
        w = jnp.where(ridx < TOPK, p_row, p_row * a_row)
        e = d * w
        rowsum = jnp.sum(e, axis=1, keepdims=True)
        xi = jnp.sum(jnp.where(lane == part, g, 0.0), axis=1, keepdims=True)
        gt = jnp.sum((g > xi).astype(jnp.float32), axis=1, keepdims=True)
        eq = jnp.sum(jnp.where((lane < part) & (g == xi), 1.0, 0.0),
                     axis=1, keepdims=True)
        hit = ((gt + eq) < TOPK).astype(jnp.float32)
        wsc = jnp.zeros((2 * TOPK, 1), jnp.float32)
        coll = jnp.zeros((2 * TOPK, 1), jnp.float32)
        for k in range(TOPK):
            lab1 = _pick_i32(labr, f1[k], lane)
            c1 = jnp.sum(jnp.where(labr == lab1, e[k:k + 1, :], 0.0))
            a1 = _pick_f32(a_row, f1[k], lane)
            wsc += jnp.where(ridx == k, a1, 0.0)
            coll += jnp.where(ridx == k, c1, 0.0)
            lab2 = _pick_i32(labr, f2[k], lane)
            c2 = jnp.sum(jnp.where(labr == lab2,
                                   e[TOPK + k:TOPK + k + 1, :], 0.0))
            p2 = _pick_f32(p_row, f2[k], lane)
            wsc += jnp.where(ridx == TOPK + k, p2, 0.0)
            coll += jnp.where(ridx == TOPK + k, c2, 0.0)
        corr = jnp.sum(hit * wsc * (rowsum - coll))
        l_acc[...] += (base + corr) / B

    @pl.when(m == P)
    def _():
        gout_ref[...] = g_acc[...]
        lout_ref[...] = l_acc[...] / P


def _align_losses(visual, textual, part_embed, attribute_embed,
                  labels, vmask, tmask):
    labc = labels.astype(jnp.int32).reshape(B, 1)
    labr = labels.astype(jnp.int32).reshape(1, B)
    pmfT = vmask.astype(jnp.float32).T
    amfT = tmask.astype(jnp.float32).T
    gout, lout = pl.pallas_call(
        _align_kernel,
        grid=(P + 1,),
        in_specs=[
            pl.BlockSpec((B, D), lambda m: (0, 0)),
            pl.BlockSpec((B, D), lambda m: (0, 0)),
            pl.BlockSpec((1, B, D), lambda m: (jnp.maximum(m - 1, 0), 0, 0)),
            pl.BlockSpec((1, B, D), lambda m: (jnp.maximum(m - 1, 0), 0, 0)),
            pl.BlockSpec((B, 1), lambda m: (0, 0)),
            pl.BlockSpec((1, B), lambda m: (0, 0)),
            pl.BlockSpec((P, B), lambda m: (0, 0)),
            pl.BlockSpec((P, B), lambda m: (0, 0)),
        ],
        out_specs=[
            pl.BlockSpec((1, 1), lambda m: (0, 0)),
            pl.BlockSpec((1, 1), lambda m: (0, 0)),
        ],
        out_shape=[
            jax.ShapeDtypeStruct((1, 1), jnp.float32),
            jax.ShapeDtypeStruct((1, 1), jnp.float32),
        ],
        scratch_shapes=[
            pltpu.VMEM((B, B), jnp.float32),
            pltpu.VMEM((B, B), jnp.float32),
            pltpu.VMEM((2 * TOPK, B), jnp.float32),
            pltpu.VMEM((1, 1), jnp.float32),
            pltpu.VMEM((1, 1), jnp.float32),
        ],
    )(visual, textual, part_embed, attribute_embed, labc, labr, pmfT, amfT)
    return gout[0, 0], lout[0, 0]


def kernel(visual_embed, textual_embed, part_embed, attribute_embed,
           seg_feat, W, labels, masks, vmask, tmask):
    inst = _instance_loss(visual_embed, textual_embed, W, labels)
    mask = _mask_loss(seg_feat, masks)
    glob, loc = _align_losses(visual_embed, textual_embed, part_embed,
                              attribute_embed, labels, vmask, tmask)
    return jnp.stack([inst, mask, glob, loc])
